# baseline (speedup 1.0000x reference)
import functools

import jax
import jax.numpy as jnp
from jax import lax
from jax.experimental import pallas as pl
from jax.experimental.pallas import tpu as pltpu
from jax.experimental.pallas import tpu_sc as plsc

N = 10000
NP = 10240
E = 320000
D = 128
H = 64
G = 64
NC = 2
NS = 16
NW = NC * NS
TPB = NP // NW
NB = NW
BCAP = 640
LISTC = NB * BCAP
E_PER = E // NW
BLKE = 2000
NBLKP = E_PER // BLKE
GCH = 128
SBLK = 2560
_SC_PARAMS = pltpu.CompilerParams(needs_layout_passes=False,
                                  use_tc_tiling_on_sc=False)

_sc_mesh = plsc.VectorSubcoreMesh(core_axis_name="c", subcore_axis_name="s")

_GDN = lax.GatherDimensionNumbers(offset_dims=(), collapsed_slice_dims=(0,),
                                  start_index_map=(0,))


def _vperm(vec, idx):
    return lax.gather(vec, idx.reshape(16, 1), _GDN, (1,),
                      mode=lax.GatherScatterMode.PROMISE_IN_BOUNDS)


@functools.partial(
    pl.kernel,
    out_type=[
        jax.ShapeDtypeStruct((NW, NB, BCAP), jnp.int32),
        jax.ShapeDtypeStruct((NW, NB, BCAP), jnp.float32),
        jax.ShapeDtypeStruct((NW, NB, BCAP), jnp.int32),
        jax.ShapeDtypeStruct((NW, 48), jnp.int32),
    ],
    mesh=_sc_mesh,
    scratch_types=[
        pltpu.VMEM((NB, BCAP), jnp.int32),
        pltpu.VMEM((NB, BCAP), jnp.float32),
        pltpu.VMEM((NB, BCAP), jnp.int32),
        pltpu.VMEM((32,), jnp.int32),
        pltpu.VMEM((48,), jnp.int32),
        pltpu.VMEM((BLKE,), jnp.int32),
        pltpu.VMEM((BLKE,), jnp.int32),
        pltpu.VMEM((BLKE,), jnp.float32),
    ],
    compiler_params=_SC_PARAMS,
)
def _sc_scatter(src_hbm, dst_hbm, ew_hbm,
                sseg_hbm, wseg_hbm, dseg_hbm, cnts_hbm,
                ssegl, wsegl, dsegl, cursv, cntb, sb, db, wb):
    cid = lax.axis_index("c")
    sid = lax.axis_index("s")
    wid = sid * NC + cid
    z16 = jnp.zeros((16,), jnp.float32)
    zi16 = jnp.zeros((16,), jnp.int32)
    iota = lax.broadcasted_iota(jnp.int32, (16,), 0)

    def zrow(r, _):
        for qq in range(BCAP // 16):
            ssegl[r, pl.ds(qq * 16, 16)] = zi16
            wsegl[r, pl.ds(qq * 16, 16)] = z16
            dsegl[r, pl.ds(qq * 16, 16)] = zi16
        return 0
    lax.fori_loop(0, NB, zrow, 0)
    cursv[pl.ds(0, 16)] = zi16
    cursv[pl.ds(16, 16)] = zi16
    cntb[pl.ds(0, 16)] = zi16
    cntb[pl.ds(16, 16)] = zi16
    cntb[pl.ds(32, 16)] = zi16

    def blk(t, _):
        base = wid * E_PER + t * BLKE
        pltpu.sync_copy(src_hbm.at[pl.ds(base, BLKE)], sb)
        pltpu.sync_copy(dst_hbm.at[pl.ds(base, BLKE)], db)
        pltpu.sync_copy(ew_hbm.at[pl.ds(base, BLKE)], wb)

        def vec(j, _):
            d = db[pl.ds(j * 16, 16)]
            s = sb[pl.ds(j * 16, 16)]
            w = wb[pl.ds(j * 16, 16)]
            b = (d * 6554) >> 21
            bs, perm = plsc.sort_key_val(b, iota)
            d_s = _vperm(d, perm)
            s_s = _vperm(s, perm)
            w_s = _vperm(w, perm)
            prev = _vperm(bs, jnp.maximum(iota - 1, 0))
            fm = (bs != prev) | (iota == 0)
            si = plsc.cummax(jnp.where(fm, iota, 0))
            rank = iota - si
            cb = plsc.load_gather(cursv, [bs])
            pos = jnp.minimum(cb + rank, BCAP - 1)
            dlc = d_s - bs * TPB
            plsc.store_scatter(ssegl, [bs, pos], s_s)
            plsc.store_scatter(wsegl, [bs, pos], w_s)
            plsc.store_scatter(dsegl, [bs, pos], dlc)
            nxt = _vperm(bs, jnp.minimum(iota + 1, 15))
            lastm = (bs != nxt) | (iota == 15)
            plsc.store_scatter(cursv, [bs], cb + rank + 1, mask=lastm)
            return 0
        lax.fori_loop(0, BLKE // 16, vec, 0)
        return 0
    lax.fori_loop(0, NBLKP, blk, 0)

    cntb[pl.ds(0, 16)] = cursv[pl.ds(0, 16)]
    cntb[pl.ds(16, 16)] = cursv[pl.ds(16, 16)]
    pltpu.sync_copy(ssegl, sseg_hbm.at[wid])
    pltpu.sync_copy(wsegl, wseg_hbm.at[wid])
    pltpu.sync_copy(dsegl, dseg_hbm.at[wid])
    pltpu.sync_copy(cntb, cnts_hbm.at[wid])


@functools.partial(
    pl.kernel,
    out_type=[
        jax.ShapeDtypeStruct((NW, LISTC), jnp.int32),
        jax.ShapeDtypeStruct((NW, LISTC), jnp.float32),
        jax.ShapeDtypeStruct((NW, LISTC), jnp.int32),
        jax.ShapeDtypeStruct((NW, 16), jnp.int32),
        jax.ShapeDtypeStruct((NP,), jnp.float32),
    ],
    mesh=_sc_mesh,
    scratch_types=[
        pltpu.VMEM((NB * BCAP,), jnp.int32),
        pltpu.VMEM((NB * BCAP,), jnp.float32),
        pltpu.VMEM((LISTC,), jnp.int32),
        pltpu.VMEM((LISTC,), jnp.float32),
        pltpu.VMEM((LISTC,), jnp.int32),
        pltpu.VMEM((NW, 48), jnp.int32),
        pltpu.VMEM((16,), jnp.int32),
        pltpu.VMEM((TPB,), jnp.float32),
        pltpu.SemaphoreType.DMA,
    ],
    compiler_params=_SC_PARAMS,
)
def _sc_build(sseg_hbm, wseg_hbm, dseg_hbm, cnts_hbm, dst_hbm, ew_hbm,
              srcc_hbm, wc_hbm, dlc_hbm, cntp_hbm, deg_hbm,
              stageI, stageF, asm_s, asm_w, asm_d, cntv, cbuf, degl, sem):
    cid = lax.axis_index("c")
    sid = lax.axis_index("s")
    wid = sid * NC + cid
    z16 = jnp.zeros((16,), jnp.float32)
    zi16 = jnp.zeros((16,), jnp.int32)

    pltpu.sync_copy(cnts_hbm, cntv)
    cts = []
    for t in range(NW):
        cv = cntv[t, pl.ds(wid, 16)]
        cts.append(cv[0])
    ctcs = [jnp.minimum(ct, BCAP) for ct in cts]
    over = cts[0] > BCAP
    for t in range(1, NW):
        over = over | (cts[t] > BCAP)
    lands = []
    run = jnp.int32(0)
    for t in range(NW):
        lands.append(run)
        run = (run + ctcs[t] + 127) & ~jnp.int32(127)
    total = run

    def zasm(i, _):
        asm_s[pl.ds(i * 16, 16)] = zi16
        asm_w[pl.ds(i * 16, 16)] = z16
        asm_d[pl.ds(i * 16, 16)] = zi16
        return 0
    lax.fori_loop(0, LISTC // 16, zasm, 0)

    def pull(seg_hbm, stage, asm):
        for t in range(NW):
            pltpu.async_copy(seg_hbm.at[t, wid],
                             stage.at[pl.ds(t * BCAP, BCAP)], sem)
        for t in range(NW):
            pltpu.make_async_copy(seg_hbm.at[t, wid],
                                  stage.at[pl.ds(t * BCAP, BCAP)], sem).wait()
        for t in range(NW):
            nv = (ctcs[t] + 15) >> 4

            def cp(i, _, t=t):
                asm[pl.ds(lands[t] + i * 16, 16)] = (
                    stage[pl.ds(t * BCAP + i * 16, 16)])
                return 0
            lax.fori_loop(0, nv, cp, 0)

    pull(wseg_hbm, stageF, asm_w)
    pull(dseg_hbm, stageI, asm_d)
    pull(sseg_hbm, stageI, asm_s)

    def zdeg(i, _):
        degl[pl.ds(i * 16, 16)] = z16
        return 0
    lax.fori_loop(0, TPB // 16, zdeg, 0)

    @pl.when(jnp.logical_not(over))
    def _degfast():
        def dacc(i, _):
            dv = asm_d[pl.ds(i * 16, 16)]
            wv = asm_w[pl.ds(i * 16, 16)]
            plsc.addupdate_scatter(degl, [dv], wv)
            return 0
        lax.fori_loop(0, LISTC // 16, dacc, 0)

    @pl.when(over)
    def _degslow():
        lo = wid * TPB

        def dblk(t, _):
            base = t * SBLK
            pltpu.sync_copy(dst_hbm.at[pl.ds(base, SBLK)],
                            stageI.at[pl.ds(0, SBLK)])
            pltpu.sync_copy(ew_hbm.at[pl.ds(base, SBLK)],
                            stageF.at[pl.ds(0, SBLK)])

            def dv16(j, _):
                d = stageI[pl.ds(j * 16, 16)]
                w = stageF[pl.ds(j * 16, 16)]
                m = (d >= lo) & (d < lo + TPB)
                plsc.addupdate_scatter(degl, [jnp.where(m, d - lo, 0)],
                                      jnp.where(m, w, 0.0))
                return 0
            lax.fori_loop(0, SBLK // 16, dv16, 0)
            return 0
        lax.fori_loop(0, E // SBLK, dblk, 0)

    cnt_out = jnp.where(over, jnp.int32(LISTC + 128), total)
    cbuf[pl.ds(0, 16)] = jnp.full((16,), cnt_out, jnp.int32)
    pltpu.sync_copy(asm_s, srcc_hbm.at[wid])
    pltpu.sync_copy(asm_w, wc_hbm.at[wid])
    pltpu.sync_copy(asm_d, dlc_hbm.at[wid])
    pltpu.sync_copy(cbuf, cntp_hbm.at[wid])
    pltpu.sync_copy(degl, deg_hbm.at[pl.ds(wid * TPB, TPB)])


@functools.partial(
    pl.kernel,
    out_type=jax.ShapeDtypeStruct((NP, H), jnp.float32),
    mesh=_sc_mesh,
    scratch_types=[
        pltpu.VMEM((LISTC,), jnp.int32),
        pltpu.VMEM((LISTC,), jnp.float32),
        pltpu.VMEM((LISTC,), jnp.int32),
        pltpu.VMEM((TPB, H), jnp.float32),
        pltpu.VMEM((4, GCH, H), jnp.bfloat16),
        pltpu.VMEM((NW, 16), jnp.int32),
        pltpu.SemaphoreType.DMA,
        pltpu.SemaphoreType.DMA,
        pltpu.SemaphoreType.DMA,
        pltpu.SemaphoreType.DMA,
    ],
    compiler_params=_SC_PARAMS,
)
def _sc_agg(g_hbm, srcs_hbm, ws_hbm, dls_hbm, cnts_hbm,
            src_hbm, dst_hbm, ew_hbm, out_hbm,
            srcl, wl, dll, acc, rows, cntv, gsem0, gsem1, gsem2, gsem3):
    cid = lax.axis_index("c")
    sid = lax.axis_index("s")
    wid = sid * NC + cid
    lo = wid * TPB
    z16 = jnp.zeros((16,), jnp.float32)
    iota = lax.broadcasted_iota(jnp.int32, (16,), 0)
    gsems = (gsem0, gsem1, gsem2, gsem3)
    lanesplat = [jnp.full((16, 1), ln, jnp.int32) for ln in range(16)]

    def _splat(vec, lane):
        return lax.gather(vec, lanesplat[lane], _GDN, (1,),
                          mode=lax.GatherScatterMode.PROMISE_IN_BOUNDS)

    pltpu.sync_copy(cnts_hbm, cntv)
    cv = cntv[wid, pl.ds(0, 16)]
    c = cv[0]
    pltpu.sync_copy(srcs_hbm.at[wid], srcl)
    pltpu.sync_copy(ws_hbm.at[wid], wl)
    pltpu.sync_copy(dls_hbm.at[wid], dll)

    def zacc(r, _):
        for j in range(H // 16):
            acc[r, pl.ds(j * 16, 16)] = z16
        return 0
    lax.fori_loop(0, TPB, zacc, 0)

    def gissue(base, buf):
        pltpu.async_copy(g_hbm.at[srcl.at[pl.ds(base, GCH)]],
                         rows.at[buf], gsems[buf])

    def gdrain(buf):
        pltpu.make_async_copy(g_hbm.at[srcl.at[pl.ds(0, GCH)]],
                              rows.at[buf], gsems[buf]).wait()

    def proc(off, buf):
        mhi = jnp.full((16,), -65536, jnp.int32)

        def kbbody(kb, _):
            o = off + kb * 16
            dv = dll[pl.ds(o, 16)]
            wv = wl[pl.ds(o, 16)]
            for lane in range(16):
                d = dv[lane]
                ws = _splat(wv, lane)
                k = kb * 16 + lane
                for half in range(H // 32):
                    v32 = rows[buf, k, pl.ds(half * 32, 32)]
                    w16 = plsc.bitcast(v32, jnp.int32)
                    lo = plsc.bitcast(w16 << 16, jnp.float32)
                    hi = plsc.bitcast(w16 & mhi, jnp.float32)
                    plsc.addupdate(acc.at[d, pl.ds(half * 32, 16)], lo * ws)
                    plsc.addupdate(acc.at[d, pl.ds(half * 32 + 16, 16)],
                                   hi * ws)
            return 0
        lax.fori_loop(0, GCH // 16, kbbody, 0)

    @pl.when(c <= LISTC)
    def _fast():
        nsub = c >> 7

        for b in range(3):
            @pl.when(b < nsub)
            def _(b=b):
                gissue(b * GCH, b)

        def quadb(p, _):
            for q in range(4):
                i = 4 * p + q

                @pl.when(i < nsub)
                def _():
                    @pl.when(i + 3 < nsub)
                    def _():
                        gissue((i + 3) * GCH, (q + 3) % 4)
                    gdrain(q)
                    proc(i * GCH, q)
            return 0
        lax.fori_loop(0, (nsub + 3) // 4, quadb, 0)

    @pl.when(c > LISTC)
    def _slow():
        def blkbody(t, _):
            base = t * SBLK
            pltpu.sync_copy(src_hbm.at[pl.ds(base, SBLK)],
                            srcl.at[pl.ds(0, SBLK)])
            pltpu.sync_copy(dst_hbm.at[pl.ds(base, SBLK)],
                            dll.at[pl.ds(0, SBLK)])
            pltpu.sync_copy(ew_hbm.at[pl.ds(base, SBLK)],
                            wl.at[pl.ds(0, SBLK)])

            def mv(j, _):
                d = dll[pl.ds(j * 16, 16)]
                s = srcl[pl.ds(j * 16, 16)]
                w = wl[pl.ds(j * 16, 16)]
                m = (d >= lo) & (d < lo + TPB)
                dll[pl.ds(j * 16, 16)] = jnp.where(m, d - lo, 0)
                srcl[pl.ds(j * 16, 16)] = jnp.where(m, s, 0)
                wl[pl.ds(j * 16, 16)] = jnp.where(m, w, 0.0)
                return 0
            lax.fori_loop(0, SBLK // 16, mv, 0)

            def sub(i2, _):
                pltpu.async_copy(g_hbm.at[srcl.at[pl.ds(i2 * GCH, GCH)]],
                                 rows.at[0], gsem0).wait()
                proc(i2 * GCH, 0)
                return 0
            lax.fori_loop(0, SBLK // GCH, sub, 0)
            return 0
        lax.fori_loop(0, E // SBLK, blkbody, 0)

    pltpu.sync_copy(acc, out_hbm.at[pl.ds(wid * TPB, TPB)])


def _tc_a_body(x_ref, w1_ref, w1t_ref, deg_ref, g_ref, gt_ref, dinv_ref):
    dinv = lax.rsqrt(deg_ref[...] + 1.0)
    dinv_ref[...] = dinv
    h = jnp.dot(x_ref[...], w1_ref[...], preferred_element_type=jnp.float32)
    g_ref[...] = (h * dinv).astype(jnp.bfloat16)
    ht = jnp.dot(x_ref[...], w1t_ref[...], preferred_element_type=jnp.float32)
    gt_ref[...] = ht * dinv


_tc_a = pl.pallas_call(
    _tc_a_body,
    out_shape=[jax.ShapeDtypeStruct((NP, H), jnp.bfloat16),
               jax.ShapeDtypeStruct((NP, H), jnp.float32),
               jax.ShapeDtypeStruct((NP, 1), jnp.float32)],
)


def _tc_b_body(acc_ref, g1t_ref, dinv_ref, b1_ref, w2t_ref, w2tt_ref,
               g2_ref, g2t_ref):
    dinv = dinv_ref[...]
    o = (acc_ref[...] + g1t_ref[...]) * dinv + b1_ref[...]
    o = jnp.maximum(o, 0.0)
    h2 = jnp.dot(o, w2t_ref[...], preferred_element_type=jnp.float32)
    g2_ref[...] = (h2 * dinv).astype(jnp.bfloat16)
    h2t = jnp.dot(o, w2tt_ref[...], preferred_element_type=jnp.float32)
    g2t_ref[...] = h2t * dinv


_tc_b = pl.pallas_call(
    _tc_b_body,
    out_shape=[jax.ShapeDtypeStruct((NP, H), jnp.bfloat16),
               jax.ShapeDtypeStruct((NP, H), jnp.float32)],
)


def _tc_c_body(acc_ref, g2_ref, dinv_ref, b2_ref, batch_ref,
               wc1_ref, bc1_ref, wc2_ref, bc2_ref, out_ref):
    o = (acc_ref[...] + g2_ref[...]) * dinv_ref[...] + b2_ref[...]
    o = jnp.maximum(o, 0.0)
    b = batch_ref[...]
    gid = lax.broadcasted_iota(jnp.int32, (G, NP), 0)
    p = (b == gid).astype(jnp.float32)
    s = jnp.dot(p, o, preferred_element_type=jnp.float32)
    cnt = jnp.sum(p, axis=1, keepdims=True)
    mean = s / jnp.maximum(cnt, 1.0)
    z = jnp.dot(mean, wc1_ref[...], preferred_element_type=jnp.float32)
    z = jnp.maximum(z + bc1_ref[...], 0.0)
    out_ref[...] = (jnp.dot(z, wc2_ref[...],
                            preferred_element_type=jnp.float32) + bc2_ref[...])


_tc_c = pl.pallas_call(
    _tc_c_body,
    out_shape=jax.ShapeDtypeStruct((G, 128), jnp.float32),
)


def kernel(x, edge_index, edge_weight, batch, W1, b1, W2, b2, Wc1, bc1, Wc2, bc2):
    src = edge_index[0]
    dst = edge_index[1]
    xp = jnp.pad(x, ((0, NP - N), (0, 0)))
    batch_p = jnp.pad(batch, (0, NP - N), constant_values=-1).reshape(1, NP)

    tau = jnp.array([*range(0, 32, 2), *range(1, 32, 2),
                     *range(32, 64, 2), *range(33, 64, 2)], dtype=jnp.int32)

    sseg, wseg, dseg, segcnt = _sc_scatter(src, dst, edge_weight)
    srcs, ws, dls, cnts, deg = _sc_build(sseg, wseg, dseg, segcnt,
                                         dst, edge_weight)
    g1, g1t, dinv = _tc_a(xp, W1, W1[:, tau], deg.reshape(NP, 1))
    acc1 = _sc_agg(g1, srcs, ws, dls, cnts, src, dst, edge_weight)
    g2, g2t = _tc_b(acc1, g1t, dinv, b1[tau].reshape(1, H),
                    W2[tau, :], W2[tau][:, tau])
    acc2 = _sc_agg(g2, srcs, ws, dls, cnts, src, dst, edge_weight)

    wc1p = jnp.pad(Wc1[tau, :], ((0, 0), (0, 128 - H // 2)))
    bc1p = jnp.pad(bc1, (0, 128 - H // 2)).reshape(1, 128)
    wc2p = jnp.pad(Wc2, ((0, 128 - H // 2), (0, 126)))
    bc2p = jnp.pad(bc2, (0, 126)).reshape(1, 128)
    outp = _tc_c(acc2, g2t, dinv, b2[tau].reshape(1, H), batch_p,
                 wc1p, bc1p, wc2p, bc2p)
    return outp[:, :2]


# speedup vs baseline: 3.1196x; 3.1196x over previous
import functools

import jax
import jax.numpy as jnp
from jax import lax
from jax.experimental import pallas as pl
from jax.experimental.pallas import tpu as pltpu
from jax.experimental.pallas import tpu_sc as plsc

N = 10000
NP = 10240
E = 320000
D = 128
H = 64
G = 64
NC = 2
NS = 16
NW = NC * NS
TPB = NP // NW
NB = NW
BCAP = 640
LISTC = NB * BCAP
E_PER = E // NW
BLKE = 2000
NBLKP = E_PER // BLKE
GCH = 128
SBLK = 2560
_SC_PARAMS = pltpu.CompilerParams(needs_layout_passes=False,
                                  use_tc_tiling_on_sc=False)

_sc_mesh = plsc.VectorSubcoreMesh(core_axis_name="c", subcore_axis_name="s")

_GDN = lax.GatherDimensionNumbers(offset_dims=(), collapsed_slice_dims=(0,),
                                  start_index_map=(0,))


def _vperm(vec, idx):
    return lax.gather(vec, idx.reshape(16, 1), _GDN, (1,),
                      mode=lax.GatherScatterMode.PROMISE_IN_BOUNDS)


@functools.partial(
    pl.kernel,
    out_type=[
        jax.ShapeDtypeStruct((NW, NB, BCAP), jnp.int32),
        jax.ShapeDtypeStruct((NW, NB, BCAP), jnp.float32),
        jax.ShapeDtypeStruct((NW, NB, BCAP), jnp.int32),
        jax.ShapeDtypeStruct((NW, 48), jnp.int32),
    ],
    mesh=_sc_mesh,
    scratch_types=[
        pltpu.VMEM((NB, BCAP), jnp.int32),
        pltpu.VMEM((NB, BCAP), jnp.float32),
        pltpu.VMEM((NB, BCAP), jnp.int32),
        pltpu.VMEM((32,), jnp.int32),
        pltpu.VMEM((48,), jnp.int32),
        pltpu.VMEM((BLKE,), jnp.int32),
        pltpu.VMEM((BLKE,), jnp.int32),
        pltpu.VMEM((BLKE,), jnp.float32),
    ],
    compiler_params=_SC_PARAMS,
)
def _sc_scatter(src_hbm, dst_hbm, ew_hbm,
                sseg_hbm, wseg_hbm, dseg_hbm, cnts_hbm,
                ssegl, wsegl, dsegl, cursv, cntb, sb, db, wb):
    cid = lax.axis_index("c")
    sid = lax.axis_index("s")
    wid = sid * NC + cid
    z16 = jnp.zeros((16,), jnp.float32)
    zi16 = jnp.zeros((16,), jnp.int32)
    iota = lax.broadcasted_iota(jnp.int32, (16,), 0)

    def zrow(r, _):
        for qq in range(BCAP // 16):
            ssegl[r, pl.ds(qq * 16, 16)] = zi16
            wsegl[r, pl.ds(qq * 16, 16)] = z16
            dsegl[r, pl.ds(qq * 16, 16)] = zi16
        return 0
    lax.fori_loop(0, NB, zrow, 0)
    cursv[pl.ds(0, 16)] = zi16
    cursv[pl.ds(16, 16)] = zi16
    cntb[pl.ds(0, 16)] = zi16
    cntb[pl.ds(16, 16)] = zi16
    cntb[pl.ds(32, 16)] = zi16

    def blk(t, _):
        base = wid * E_PER + t * BLKE
        pltpu.sync_copy(src_hbm.at[pl.ds(base, BLKE)], sb)
        pltpu.sync_copy(dst_hbm.at[pl.ds(base, BLKE)], db)
        pltpu.sync_copy(ew_hbm.at[pl.ds(base, BLKE)], wb)

        def vec(j, _):
            d = db[pl.ds(j * 16, 16)]
            s = sb[pl.ds(j * 16, 16)]
            w = wb[pl.ds(j * 16, 16)]
            b = (d * 6554) >> 21
            bs, perm = plsc.sort_key_val(b, iota)
            d_s = _vperm(d, perm)
            s_s = _vperm(s, perm)
            w_s = _vperm(w, perm)
            prev = _vperm(bs, jnp.maximum(iota - 1, 0))
            fm = (bs != prev) | (iota == 0)
            si = plsc.cummax(jnp.where(fm, iota, 0))
            rank = iota - si
            cb = plsc.load_gather(cursv, [bs])
            pos = jnp.minimum(cb + rank, BCAP - 1)
            dlc = d_s - bs * TPB
            plsc.store_scatter(ssegl, [bs, pos], s_s)
            plsc.store_scatter(wsegl, [bs, pos], w_s)
            plsc.store_scatter(dsegl, [bs, pos], dlc)
            nxt = _vperm(bs, jnp.minimum(iota + 1, 15))
            lastm = (bs != nxt) | (iota == 15)
            plsc.store_scatter(cursv, [bs], cb + rank + 1, mask=lastm)
            return 0
        lax.fori_loop(0, BLKE // 16, vec, 0)
        return 0
    lax.fori_loop(0, NBLKP, blk, 0)

    cntb[pl.ds(0, 16)] = cursv[pl.ds(0, 16)]
    cntb[pl.ds(16, 16)] = cursv[pl.ds(16, 16)]
    pltpu.sync_copy(ssegl, sseg_hbm.at[wid])
    pltpu.sync_copy(wsegl, wseg_hbm.at[wid])
    pltpu.sync_copy(dsegl, dseg_hbm.at[wid])
    pltpu.sync_copy(cntb, cnts_hbm.at[wid])


@functools.partial(
    pl.kernel,
    out_type=[
        jax.ShapeDtypeStruct((NW, LISTC), jnp.int32),
        jax.ShapeDtypeStruct((NW, LISTC), jnp.float32),
        jax.ShapeDtypeStruct((NW, LISTC), jnp.int32),
        jax.ShapeDtypeStruct((NW, 16), jnp.int32),
        jax.ShapeDtypeStruct((NP,), jnp.float32),
    ],
    mesh=_sc_mesh,
    scratch_types=[
        pltpu.VMEM((NB * BCAP,), jnp.int32),
        pltpu.VMEM((NB * BCAP,), jnp.float32),
        pltpu.VMEM((LISTC,), jnp.int32),
        pltpu.VMEM((LISTC,), jnp.float32),
        pltpu.VMEM((LISTC,), jnp.int32),
        pltpu.VMEM((NW, 48), jnp.int32),
        pltpu.VMEM((16,), jnp.int32),
        pltpu.VMEM((TPB,), jnp.float32),
        pltpu.SemaphoreType.DMA,
    ],
    compiler_params=_SC_PARAMS,
)
def _sc_build(sseg_hbm, wseg_hbm, dseg_hbm, cnts_hbm, dst_hbm, ew_hbm,
              srcc_hbm, wc_hbm, dlc_hbm, cntp_hbm, deg_hbm,
              stageI, stageF, asm_s, asm_w, asm_d, cntv, cbuf, degl, sem):
    cid = lax.axis_index("c")
    sid = lax.axis_index("s")
    wid = sid * NC + cid
    z16 = jnp.zeros((16,), jnp.float32)
    zi16 = jnp.zeros((16,), jnp.int32)

    pltpu.sync_copy(cnts_hbm, cntv)
    cts = []
    for t in range(NW):
        cv = cntv[t, pl.ds(wid, 16)]
        cts.append(cv[0])
    ctcs = [jnp.minimum(ct, BCAP) for ct in cts]
    over = cts[0] > BCAP
    for t in range(1, NW):
        over = over | (cts[t] > BCAP)
    lands = []
    run = jnp.int32(0)
    for t in range(NW):
        lands.append(run)
        run = (run + ctcs[t] + 15) & ~jnp.int32(15)
    total = run

    iota = lax.broadcasted_iota(jnp.int32, (16,), 0)

    def zasm(i, _):
        asm_s[pl.ds(i * 16, 16)] = (i * 16 + iota) & 8191
        asm_w[pl.ds(i * 16, 16)] = z16
        asm_d[pl.ds(i * 16, 16)] = zi16
        return 0
    lax.fori_loop(0, LISTC // 16, zasm, 0)

    def pull(seg_hbm, stage, asm):
        for t in range(NW):
            pltpu.async_copy(seg_hbm.at[t, wid],
                             stage.at[pl.ds(t * BCAP, BCAP)], sem)
        for t in range(NW):
            pltpu.make_async_copy(seg_hbm.at[t, wid],
                                  stage.at[pl.ds(t * BCAP, BCAP)], sem).wait()
        for t in range(NW):
            nv = (ctcs[t] + 15) >> 4

            def cp(i, _, t=t):
                asm[pl.ds(lands[t] + i * 16, 16)] = (
                    stage[pl.ds(t * BCAP + i * 16, 16)])
                return 0
            lax.fori_loop(0, nv, cp, 0)

    pull(wseg_hbm, stageF, asm_w)
    pull(dseg_hbm, stageI, asm_d)
    pull(sseg_hbm, stageI, asm_s)

    def zdeg(i, _):
        degl[pl.ds(i * 16, 16)] = z16
        return 0
    lax.fori_loop(0, TPB // 16, zdeg, 0)

    @pl.when(jnp.logical_not(over))
    def _degfast():
        def dacc(i, _):
            dv = asm_d[pl.ds(i * 16, 16)]
            wv = asm_w[pl.ds(i * 16, 16)]
            plsc.addupdate_scatter(degl, [dv], wv)
            return 0
        lax.fori_loop(0, LISTC // 16, dacc, 0)

    @pl.when(over)
    def _degslow():
        lo = wid * TPB

        def dblk(t, _):
            base = t * SBLK
            pltpu.sync_copy(dst_hbm.at[pl.ds(base, SBLK)],
                            stageI.at[pl.ds(0, SBLK)])
            pltpu.sync_copy(ew_hbm.at[pl.ds(base, SBLK)],
                            stageF.at[pl.ds(0, SBLK)])

            def dv16(j, _):
                d = stageI[pl.ds(j * 16, 16)]
                w = stageF[pl.ds(j * 16, 16)]
                m = (d >= lo) & (d < lo + TPB)
                plsc.addupdate_scatter(degl, [jnp.where(m, d - lo, 0)],
                                      jnp.where(m, w, 0.0))
                return 0
            lax.fori_loop(0, SBLK // 16, dv16, 0)
            return 0
        lax.fori_loop(0, E // SBLK, dblk, 0)

    cnt_out = jnp.where(over, jnp.int32(LISTC + 128), total)
    cbuf[pl.ds(0, 16)] = jnp.full((16,), cnt_out, jnp.int32)
    pltpu.sync_copy(asm_s, srcc_hbm.at[wid])
    pltpu.sync_copy(asm_w, wc_hbm.at[wid])
    pltpu.sync_copy(asm_d, dlc_hbm.at[wid])
    pltpu.sync_copy(cbuf, cntp_hbm.at[wid])
    pltpu.sync_copy(degl, deg_hbm.at[pl.ds(wid * TPB, TPB)])


@functools.partial(
    pl.kernel,
    out_type=jax.ShapeDtypeStruct((NP, H), jnp.float32),
    mesh=_sc_mesh,
    scratch_types=[
        pltpu.VMEM((LISTC,), jnp.int32),
        pltpu.VMEM((LISTC,), jnp.float32),
        pltpu.VMEM((LISTC,), jnp.int32),
        pltpu.VMEM((TPB, H), jnp.float32),
        pltpu.VMEM((4, GCH, H), jnp.bfloat16),
        pltpu.VMEM((NW, 16), jnp.int32),
        pltpu.SemaphoreType.DMA,
        pltpu.SemaphoreType.DMA,
        pltpu.SemaphoreType.DMA,
        pltpu.SemaphoreType.DMA,
    ],
    compiler_params=_SC_PARAMS,
)
def _sc_agg(g_hbm, srcs_hbm, ws_hbm, dls_hbm, cnts_hbm,
            src_hbm, dst_hbm, ew_hbm, out_hbm,
            srcl, wl, dll, acc, rows, cntv, gsem0, gsem1, gsem2, gsem3):
    cid = lax.axis_index("c")
    sid = lax.axis_index("s")
    wid = sid * NC + cid
    lo = wid * TPB
    z16 = jnp.zeros((16,), jnp.float32)
    iota = lax.broadcasted_iota(jnp.int32, (16,), 0)
    gsems = (gsem0, gsem1, gsem2, gsem3)
    lanesplat = [jnp.full((16, 1), ln, jnp.int32) for ln in range(16)]

    def _splat(vec, lane):
        return lax.gather(vec, lanesplat[lane], _GDN, (1,),
                          mode=lax.GatherScatterMode.PROMISE_IN_BOUNDS)

    pltpu.sync_copy(cnts_hbm, cntv)
    cv = cntv[wid, pl.ds(0, 16)]
    c = cv[0]
    pltpu.sync_copy(srcs_hbm.at[wid], srcl)
    pltpu.sync_copy(ws_hbm.at[wid], wl)
    pltpu.sync_copy(dls_hbm.at[wid], dll)

    def zacc(r, _):
        for j in range(H // 16):
            acc[r, pl.ds(j * 16, 16)] = z16
        return 0
    lax.fori_loop(0, TPB, zacc, 0)

    def gissue(base, buf):
        pltpu.async_copy(g_hbm.at[srcl.at[pl.ds(base, GCH)]],
                         rows.at[buf], gsems[buf])

    def gdrain(buf):
        pltpu.make_async_copy(g_hbm.at[srcl.at[pl.ds(0, GCH)]],
                              rows.at[buf], gsems[buf]).wait()

    def proc(off, buf):
        mhi = jnp.full((16,), -65536, jnp.int32)

        def kbbody(kb, _):
            o = off + kb * 16
            dv = dll[pl.ds(o, 16)]
            wv = wl[pl.ds(o, 16)]
            for lane in range(16):
                d = dv[lane]
                ws = _splat(wv, lane)
                k = kb * 16 + lane
                for half in range(H // 32):
                    v32 = rows[buf, k, pl.ds(half * 32, 32)]
                    w16 = plsc.bitcast(v32, jnp.int32)
                    lo = plsc.bitcast(w16 << 16, jnp.float32)
                    hi = plsc.bitcast(w16 & mhi, jnp.float32)
                    plsc.addupdate(acc.at[d, pl.ds(half * 32, 16)], lo * ws)
                    plsc.addupdate(acc.at[d, pl.ds(half * 32 + 16, 16)],
                                   hi * ws)
            return 0
        lax.fori_loop(0, GCH // 16, kbbody, 0)

    @pl.when(c <= LISTC)
    def _fast():
        nsub = (c + GCH - 1) >> 7

        for b in range(3):
            @pl.when(b < nsub)
            def _(b=b):
                gissue(b * GCH, b)

        def quadb(p, _):
            for q in range(4):
                i = 4 * p + q

                @pl.when(i < nsub)
                def _():
                    @pl.when(i + 3 < nsub)
                    def _():
                        gissue((i + 3) * GCH, (q + 3) % 4)
                    gdrain(q)
                    proc(i * GCH, q)
            return 0
        lax.fori_loop(0, (nsub + 3) // 4, quadb, 0)

    @pl.when(c > LISTC)
    def _slow():
        def blkbody(t, _):
            base = t * SBLK
            pltpu.sync_copy(src_hbm.at[pl.ds(base, SBLK)],
                            srcl.at[pl.ds(0, SBLK)])
            pltpu.sync_copy(dst_hbm.at[pl.ds(base, SBLK)],
                            dll.at[pl.ds(0, SBLK)])
            pltpu.sync_copy(ew_hbm.at[pl.ds(base, SBLK)],
                            wl.at[pl.ds(0, SBLK)])

            def mv(j, _):
                d = dll[pl.ds(j * 16, 16)]
                s = srcl[pl.ds(j * 16, 16)]
                w = wl[pl.ds(j * 16, 16)]
                m = (d >= lo) & (d < lo + TPB)
                dll[pl.ds(j * 16, 16)] = jnp.where(m, d - lo, 0)
                srcl[pl.ds(j * 16, 16)] = jnp.where(m, s, 0)
                wl[pl.ds(j * 16, 16)] = jnp.where(m, w, 0.0)
                return 0
            lax.fori_loop(0, SBLK // 16, mv, 0)

            def sub(i2, _):
                pltpu.async_copy(g_hbm.at[srcl.at[pl.ds(i2 * GCH, GCH)]],
                                 rows.at[0], gsem0).wait()
                proc(i2 * GCH, 0)
                return 0
            lax.fori_loop(0, SBLK // GCH, sub, 0)
            return 0
        lax.fori_loop(0, E // SBLK, blkbody, 0)

    pltpu.sync_copy(acc, out_hbm.at[pl.ds(wid * TPB, TPB)])


def _tc_a_body(x_ref, w1_ref, w1t_ref, deg_ref, g_ref, gt_ref, dinv_ref):
    dinv = lax.rsqrt(deg_ref[...] + 1.0)
    dinv_ref[...] = dinv
    h = jnp.dot(x_ref[...], w1_ref[...], preferred_element_type=jnp.float32)
    g_ref[...] = (h * dinv).astype(jnp.bfloat16)
    ht = jnp.dot(x_ref[...], w1t_ref[...], preferred_element_type=jnp.float32)
    gt_ref[...] = ht * dinv


_tc_a = pl.pallas_call(
    _tc_a_body,
    out_shape=[jax.ShapeDtypeStruct((NP, H), jnp.bfloat16),
               jax.ShapeDtypeStruct((NP, H), jnp.float32),
               jax.ShapeDtypeStruct((NP, 1), jnp.float32)],
)


def _tc_b_body(acc_ref, g1t_ref, dinv_ref, b1_ref, w2t_ref, w2tt_ref,
               g2_ref, g2t_ref):
    dinv = dinv_ref[...]
    o = (acc_ref[...] + g1t_ref[...]) * dinv + b1_ref[...]
    o = jnp.maximum(o, 0.0)
    h2 = jnp.dot(o, w2t_ref[...], preferred_element_type=jnp.float32)
    g2_ref[...] = (h2 * dinv).astype(jnp.bfloat16)
    h2t = jnp.dot(o, w2tt_ref[...], preferred_element_type=jnp.float32)
    g2t_ref[...] = h2t * dinv


_tc_b = pl.pallas_call(
    _tc_b_body,
    out_shape=[jax.ShapeDtypeStruct((NP, H), jnp.bfloat16),
               jax.ShapeDtypeStruct((NP, H), jnp.float32)],
)


def _tc_c_body(acc_ref, g2_ref, dinv_ref, b2_ref, batch_ref,
               wc1_ref, bc1_ref, wc2_ref, bc2_ref, out_ref):
    o = (acc_ref[...] + g2_ref[...]) * dinv_ref[...] + b2_ref[...]
    o = jnp.maximum(o, 0.0)
    b = batch_ref[...]
    gid = lax.broadcasted_iota(jnp.int32, (G, NP), 0)
    p = (b == gid).astype(jnp.float32)
    s = jnp.dot(p, o, preferred_element_type=jnp.float32)
    cnt = jnp.sum(p, axis=1, keepdims=True)
    mean = s / jnp.maximum(cnt, 1.0)
    z = jnp.dot(mean, wc1_ref[...], preferred_element_type=jnp.float32)
    z = jnp.maximum(z + bc1_ref[...], 0.0)
    out_ref[...] = (jnp.dot(z, wc2_ref[...],
                            preferred_element_type=jnp.float32) + bc2_ref[...])


_tc_c = pl.pallas_call(
    _tc_c_body,
    out_shape=jax.ShapeDtypeStruct((G, 128), jnp.float32),
)


def kernel(x, edge_index, edge_weight, batch, W1, b1, W2, b2, Wc1, bc1, Wc2, bc2):
    src = edge_index[0]
    dst = edge_index[1]
    xp = jnp.pad(x, ((0, NP - N), (0, 0)))
    batch_p = jnp.pad(batch, (0, NP - N), constant_values=-1).reshape(1, NP)

    tau = jnp.array([*range(0, 32, 2), *range(1, 32, 2),
                     *range(32, 64, 2), *range(33, 64, 2)], dtype=jnp.int32)

    sseg, wseg, dseg, segcnt = _sc_scatter(src, dst, edge_weight)
    srcs, ws, dls, cnts, deg = _sc_build(sseg, wseg, dseg, segcnt,
                                         dst, edge_weight)
    g1, g1t, dinv = _tc_a(xp, W1, W1[:, tau], deg.reshape(NP, 1))
    acc1 = _sc_agg(g1, srcs, ws, dls, cnts, src, dst, edge_weight)
    g2, g2t = _tc_b(acc1, g1t, dinv, b1[tau].reshape(1, H),
                    W2[tau, :], W2[tau][:, tau])
    acc2 = _sc_agg(g2, srcs, ws, dls, cnts, src, dst, edge_weight)

    wc1p = jnp.pad(Wc1[tau, :], ((0, 0), (0, 128 - H // 2)))
    bc1p = jnp.pad(bc1, (0, 128 - H // 2)).reshape(1, 128)
    wc2p = jnp.pad(Wc2, ((0, 128 - H // 2), (0, 126)))
    bc2p = jnp.pad(bc2, (0, 126)).reshape(1, 128)
    outp = _tc_c(acc2, g2t, dinv, b2[tau].reshape(1, H), batch_p,
                 wc1p, bc1p, wc2p, bc2p)
    return outp[:, :2]
